# Initial kernel scaffold; baseline (speedup 1.0000x reference)
import jax
import jax.numpy as jnp
from jax.experimental import pallas as pl


def kernel(gene_set_features, attention_weights):
    raise NotImplementedError("write your pallas kernel here")


# speedup vs baseline: 4.6639x; 4.6639x over previous
import jax
import jax.numpy as jnp
from jax.experimental import pallas as pl

B = 16384
G = 16
K = 64
F = G * K
BB = 2048


def _body(w_ref, x_ref, o_ref):
    w = w_ref[:]
    row_grp = jax.lax.broadcasted_iota(jnp.int32, (F, G), 0) // K
    col = jax.lax.broadcasted_iota(jnp.int32, (F, G), 1)
    mask = row_grp == col
    wb = jnp.where(mask, w, -jnp.inf)
    gm = jnp.max(wb, axis=0, keepdims=True)
    e = jnp.exp(wb - gm)
    Wm = e / jnp.sum(e, axis=0, keepdims=True)
    o_ref[:] = jnp.dot(x_ref[:], Wm, preferred_element_type=jnp.float32)


def kernel(gene_set_features, attention_weights):
    wcol = attention_weights.reshape(F, 1)
    return pl.pallas_call(
        _body,
        grid=(B // BB,),
        in_specs=[
            pl.BlockSpec((F, 1), lambda i: (0, 0)),
            pl.BlockSpec((BB, F), lambda i: (i, 0)),
        ],
        out_specs=pl.BlockSpec((BB, G), lambda i: (i, 0)),
        out_shape=jax.ShapeDtypeStruct((B, G), jnp.float32),
    )(wcol, gene_set_features)
